# baseline (speedup 1.0000x reference)
import functools

import jax
import jax.numpy as jnp
from jax import lax
from jax.experimental import pallas as pl
from jax.experimental.pallas import tpu as pltpu
from jax.experimental.pallas import tpu_sc as plsc

N_NODES = 10000
N_EDGES = 160000
NREL = 8
NBASES = 30

NTILES = 32
GPN = 16
NGROUPS = 640
NPAD = NGROUPS * GPN
GPT = NGROUPS // NTILES
ACCR = GPN * NREL
KP = 48
NBCH = 8
CHE = NBCH * KP
EPAD = 160512
GOFFPAD = 656
ND = 256
NBLK = NPAD // ND


def _sc_mesh():
    return plsc.VectorSubcoreMesh(core_axis_name="c", subcore_axis_name="s")


def _agg_body(din, with_cnt=False):
    def body(*refs):
        if with_cnt:
            (h, srcp, lidxp, goffs, zeros, zeros16, agg, cnt,
             srcv, lidxv, stage0, stage1, acc, accc, offv,
             sem0, sem1) = refs
        else:
            (h, srcp, lidxp, goffs, zeros, agg,
             srcv, lidxv, stage0, stage1, acc, offv, sem0, sem1) = refs
            accc = zeros16 = cnt = None
        ones = jnp.full((16,), 1.0, jnp.float32)
        c = lax.axis_index("c")
        s = lax.axis_index("s")
        w = c * 16 + s

        pltpu.sync_copy(goffs, offv)
        pltpu.sync_copy(zeros, acc)
        if with_cnt:
            pltpu.sync_copy(zeros16, accc)
        stages = (stage0, stage1)
        sems = (sem0, sem1)

        def read_off(i):
            return offv[pl.ds(i, 16)][0]

        def group_step(j, lo):
            g = w * GPT + j
            hi = read_off(g + 1)
            lo_al = (lo // KP) * KP
            nb = (hi - lo_al + KP - 1) // KP
            nc = (nb + NBCH - 1) // NBCH

            def chunk_step(ci, _):
                cbase = lo_al + ci * CHE
                nbc = jnp.minimum(nb - ci * NBCH, NBCH)
                pltpu.sync_copy(srcp.at[pl.ds(cbase, CHE)], srcv)
                pltpu.sync_copy(lidxp.at[pl.ds(cbase, CHE)],
                                lidxv.at[pl.ds(0, CHE)])

                def idxr(b):
                    return srcv.at[pl.ds(b * KP, KP)]

                @pl.when(nbc > 0)
                def _():
                    pltpu.async_copy(h.at[idxr(0)], stages[0], sems[0])

                for b in range(NBCH):
                    if b + 1 < NBCH:
                        @pl.when(b + 1 < nbc)
                        def _(b=b):
                            pltpu.async_copy(h.at[idxr(b + 1)],
                                             stages[(b + 1) % 2],
                                             sems[(b + 1) % 2])

                    @pl.when(b < nbc)
                    def _(b=b):
                        pltpu.make_async_copy(h.at[idxr(b)],
                                              stages[b % 2],
                                              sems[b % 2]).wait()
                        bb = cbase + b * KP
                        e0 = jnp.maximum(lo - bb, 0)
                        e1 = jnp.minimum(hi - bb, KP)
                        stg = stages[b % 2]

                        @plsc.parallel_loop(e0, e1, 1, unroll=2)
                        def edge(e):
                            lid = lidxv[pl.ds(b * KP + e, 16)][0]
                            base = lid * din
                            if with_cnt:
                                plsc.addupdate(
                                    accc.at[pl.ds(lid * 16, 16)], ones)
                            for ch in range(din // 16):
                                v = stg[e, pl.ds(ch * 16, 16)]
                                plsc.addupdate(
                                    acc.at[pl.ds(base + ch * 16, 16)], v)
                return 0

            lax.fori_loop(0, nc, chunk_step, 0)
            pltpu.sync_copy(acc, agg.at[pl.ds(g * ACCR * din, ACCR * din)])
            pltpu.sync_copy(zeros, acc)
            if with_cnt:
                pltpu.sync_copy(accc,
                                cnt.at[pl.ds(g * ACCR * 16, ACCR * 16)])
                pltpu.sync_copy(zeros16, accc)
            return hi

        lo0 = read_off(w * GPT)
        lax.fori_loop(0, GPT, group_step, lo0)

    return body


def _cnt_body():
    def body(lidxp, goffs, zeros, agg, lidxv, acc, offv, sem):
        c = lax.axis_index("c")
        s = lax.axis_index("s")
        w = c * 16 + s

        pltpu.sync_copy(goffs, offv)
        pltpu.sync_copy(zeros, acc)
        ones = jnp.full((16,), 1.0, jnp.float32)

        def read_off(i):
            return offv[pl.ds(i, 16)][0]

        def group_step(j, lo):
            g = w * GPT + j
            hi = read_off(g + 1)
            lo_al = (lo // 8) * 8
            nb = (hi - lo_al + CHE - 1) // CHE

            def batch(bi, _):
                abase = lo_al + bi * CHE
                pltpu.sync_copy(lidxp.at[pl.ds(abase, CHE)],
                                lidxv.at[pl.ds(0, CHE)])
                e0 = jnp.maximum(lo - abase, 0)
                e1 = jnp.minimum(hi - abase, CHE)

                def edge(e, _):
                    lid = lidxv[pl.ds(e, 16)][0]
                    plsc.addupdate(acc.at[pl.ds(lid * 16, 16)], ones)
                    return 0

                lax.fori_loop(e0, e1, edge, 0)
                return 0

            lax.fori_loop(0, nb, batch, 0)
            pltpu.sync_copy(acc, agg.at[pl.ds(g * ACCR * 16, ACCR * 16)])
            pltpu.sync_copy(zeros, acc)
            return hi

        lo0 = read_off(w * GPT)
        lax.fori_loop(0, GPT, group_step, lo0)

    return body


@functools.cache
def _agg_fn(din, with_cnt=False):
    scratch = [
        pltpu.VMEM((CHE,), jnp.int32),
        pltpu.VMEM((CHE + 16,), jnp.int32),
        pltpu.VMEM((KP, din), jnp.float32),
        pltpu.VMEM((KP, din), jnp.float32),
        pltpu.VMEM((ACCR * din,), jnp.float32),
    ]
    out_type = jax.ShapeDtypeStruct((NPAD * NREL * din,), jnp.float32)
    if with_cnt:
        scratch.append(pltpu.VMEM((ACCR * 16,), jnp.float32))
        out_type = (out_type,
                    jax.ShapeDtypeStruct((NPAD * NREL * 16,), jnp.float32))
    scratch += [
        pltpu.VMEM((GOFFPAD,), jnp.int32),
        pltpu.SemaphoreType.DMA,
        pltpu.SemaphoreType.DMA,
    ]
    return pl.kernel(
        _agg_body(din, with_cnt),
        out_type=out_type,
        mesh=_sc_mesh(),
        scratch_types=scratch,
        name=f"rgcn_sc_agg_{din}" + ("_cnt" if with_cnt else ""),
    )


@functools.cache
def _cnt_fn():
    scratch = [
        pltpu.VMEM((CHE + 16,), jnp.int32),
        pltpu.VMEM((ACCR * 16,), jnp.float32),
        pltpu.VMEM((GOFFPAD,), jnp.int32),
        pltpu.SemaphoreType.DMA,
    ]
    return pl.kernel(
        _cnt_body(),
        out_type=jax.ShapeDtypeStruct((NPAD * NREL * 16,), jnp.float32),
        mesh=_sc_mesh(),
        scratch_types=scratch,
        name="rgcn_sc_cnt",
    )


@functools.cache
def _w_fn(din, dout):
    cols = din * dout
    bw = 8192

    def body(comp_ref, basis_ref, o_ref):
        o_ref[...] = jnp.dot(comp_ref[...], basis_ref[...],
                             preferred_element_type=jnp.float32)

    return pl.pallas_call(
        body,
        grid=(cols // bw,),
        in_specs=[
            pl.BlockSpec((NREL, NBASES), lambda j: (0, 0)),
            pl.BlockSpec((NBASES, bw), lambda j: (0, j)),
        ],
        out_specs=pl.BlockSpec((NREL, bw), lambda j: (0, j)),
        out_shape=jax.ShapeDtypeStruct((NREL, cols), jnp.float32),
        name=f"rgcn_tc_w_{din}_{dout}",
    )


@functools.cache
def _main_fn(din, dout):
    def body(h_ref, root_ref, w_ref, agg_ref, cnt_ref, bias_ref, o_ref):
        acc = jnp.dot(h_ref[...].astype(jnp.bfloat16),
                      root_ref[...].astype(jnp.bfloat16),
                      preferred_element_type=jnp.float32)
        parts = []
        for r in range(NREL):
            inv = 1.0 / jnp.maximum(cnt_ref[:, r * 16:r * 16 + 1], 1.0)
            parts.append(agg_ref[:, r * din:(r + 1) * din] * inv)
        sa = jnp.concatenate(parts, axis=1)
        acc = acc + jnp.dot(sa.astype(jnp.bfloat16),
                            w_ref[...].astype(jnp.bfloat16),
                            preferred_element_type=jnp.float32)
        o_ref[...] = jnp.maximum(acc + bias_ref[...], 0.0)

    return pl.pallas_call(
        body,
        grid=(NBLK,),
        in_specs=[
            pl.BlockSpec((ND, din), lambda i: (i, 0)),
            pl.BlockSpec((din, dout), lambda i: (0, 0)),
            pl.BlockSpec((NREL * din, dout), lambda i: (0, 0)),
            pl.BlockSpec((ND, NREL * din), lambda i: (i, 0)),
            pl.BlockSpec((ND, NREL * 16), lambda i: (i, 0)),
            pl.BlockSpec((1, dout), lambda i: (0, 0)),
        ],
        out_specs=pl.BlockSpec((ND, dout), lambda i: (i, 0)),
        out_shape=jax.ShapeDtypeStruct((NPAD, dout), jnp.float32),
        name=f"rgcn_tc_main_{din}_{dout}",
    )


def kernel(x, edge_index, edge_type,
           basis0, comp0, root0, bias0,
           basis1, comp1, root1, bias1,
           basis2, comp2, root2, bias2):
    src = edge_index[0]
    dst = edge_index[1]
    key = jnp.left_shift(dst.astype(jnp.uint32), 18) | jnp.arange(
        N_EDGES, dtype=jnp.uint32)
    key = jnp.sort(key)
    order = jnp.bitwise_and(key, (1 << 18) - 1).astype(jnp.int32)
    dst_s = jnp.right_shift(key, 18).astype(jnp.int32)
    src_s = src[order].astype(jnp.int32)
    et_s = edge_type[order].astype(jnp.int32)
    lidx = jnp.bitwise_and(dst_s, GPN - 1) * NREL + et_s
    pad = EPAD - N_EDGES
    srcp = jnp.concatenate([src_s, jnp.zeros((pad,), jnp.int32)])
    lidxp = jnp.concatenate([lidx, jnp.zeros((pad,), jnp.int32)])
    goffs = jnp.searchsorted(
        dst_s,
        jnp.arange(NGROUPS + 1, dtype=jnp.int32) * GPN).astype(jnp.int32)
    goffs = jnp.concatenate(
        [goffs, jnp.zeros((GOFFPAD - NGROUPS - 1,), jnp.int32)])

    h = jnp.pad(x, ((0, NPAD - N_NODES), (0, 0)))
    params = [(basis0, comp0, root0, bias0),
              (basis1, comp1, root1, bias1),
              (basis2, comp2, root2, bias2)]
    cnt = None
    for li, (basis, comp, root, bias) in enumerate(params):
        din, dout = basis.shape[1], basis.shape[2]
        if li == 0:
            aggflat, cntflat = _agg_fn(din, True)(
                h, srcp, lidxp, goffs,
                jnp.zeros((ACCR * din,), jnp.float32),
                jnp.zeros((ACCR * 16,), jnp.float32))
            cnt = cntflat.reshape(NPAD, NREL * 16)
        else:
            aggflat = _agg_fn(din)(h, srcp, lidxp, goffs,
                                   jnp.zeros((ACCR * din,), jnp.float32))
        agg = aggflat.reshape(NPAD, NREL * din)
        w2 = _w_fn(din, dout)(comp, basis.reshape(NBASES, din * dout))
        wstk = w2.reshape(NREL * din, dout)
        h = _main_fn(din, dout)(h, root, wstk, agg, cnt,
                                bias.reshape(1, dout))
    return h[:N_NODES]


# speedup vs baseline: 1.0120x; 1.0120x over previous
import functools

import jax
import jax.numpy as jnp
from jax import lax
from jax.experimental import pallas as pl
from jax.experimental.pallas import tpu as pltpu
from jax.experimental.pallas import tpu_sc as plsc

N_NODES = 10000
N_EDGES = 160000
NREL = 8
NBASES = 30

NTILES = 32
GPN = 16
NGROUPS = 640
NPAD = NGROUPS * GPN
GPT = NGROUPS // NTILES
ACCR = GPN * NREL
KP = 48
NBCH = 8
CHE = NBCH * KP
EPAD = 160512
GOFFPAD = 656
ND = 256
NBLK = NPAD // ND


def _sc_mesh():
    return plsc.VectorSubcoreMesh(core_axis_name="c", subcore_axis_name="s")


def _agg_body(din, with_cnt=False):
    def body(*refs):
        if with_cnt:
            (h, srcp, lidxp, goffs, zeros, zeros16, agg, cnt,
             srcv, lidxv, stage0, stage1, acc, accc, offv,
             sem0, sem1) = refs
        else:
            (h, srcp, lidxp, goffs, zeros, agg,
             srcv, lidxv, stage0, stage1, acc, offv, sem0, sem1) = refs
            accc = zeros16 = cnt = None
        ones = jnp.full((16,), 1.0, jnp.float32)
        c = lax.axis_index("c")
        s = lax.axis_index("s")
        w = c * 16 + s

        pltpu.sync_copy(goffs, offv)
        pltpu.sync_copy(zeros, acc)
        if with_cnt:
            pltpu.sync_copy(zeros16, accc)
        stages = (stage0, stage1)
        sems = (sem0, sem1)

        def read_off(i):
            return offv[pl.ds(i, 16)][0]

        def group_step(j, lo):
            g = w * GPT + j
            hi = read_off(g + 1)
            lo_al = (lo // KP) * KP
            nb = (hi - lo_al + KP - 1) // KP
            nc = (nb + NBCH - 1) // NBCH

            def chunk_step(ci, _):
                cbase = lo_al + ci * CHE
                nbc = jnp.minimum(nb - ci * NBCH, NBCH)
                pltpu.sync_copy(srcp.at[pl.ds(cbase, CHE)], srcv)
                pltpu.sync_copy(lidxp.at[pl.ds(cbase, CHE)],
                                lidxv.at[pl.ds(0, CHE)])

                def idxr(b):
                    return srcv.at[pl.ds(b * KP, KP)]

                @pl.when(nbc > 0)
                def _():
                    pltpu.async_copy(h.at[idxr(0)], stages[0], sems[0])

                for b in range(NBCH):
                    if b + 1 < NBCH:
                        @pl.when(b + 1 < nbc)
                        def _(b=b):
                            pltpu.async_copy(h.at[idxr(b + 1)],
                                             stages[(b + 1) % 2],
                                             sems[(b + 1) % 2])

                    @pl.when(b < nbc)
                    def _(b=b):
                        pltpu.make_async_copy(h.at[idxr(b)],
                                              stages[b % 2],
                                              sems[b % 2]).wait()
                        bb = cbase + b * KP
                        e0 = jnp.maximum(lo - bb, 0)
                        e1 = jnp.minimum(hi - bb, KP)
                        stg = stages[b % 2]

                        @plsc.parallel_loop(e0, e1, 1, unroll=2)
                        def edge(e):
                            lid = lidxv[pl.ds(b * KP + e, 16)][0]
                            base = lid * din
                            if with_cnt:
                                plsc.addupdate(
                                    accc.at[pl.ds(lid * 16, 16)], ones)
                            for ch in range(din // 16):
                                v = stg[e, pl.ds(ch * 16, 16)]
                                plsc.addupdate(
                                    acc.at[pl.ds(base + ch * 16, 16)], v)
                return 0

            lax.fori_loop(0, nc, chunk_step, 0)
            pltpu.sync_copy(acc, agg.at[pl.ds(g * ACCR * din, ACCR * din)])
            pltpu.sync_copy(zeros, acc)
            if with_cnt:
                pltpu.sync_copy(accc,
                                cnt.at[pl.ds(g * ACCR * 16, ACCR * 16)])
                pltpu.sync_copy(zeros16, accc)
            return hi

        lo0 = read_off(w * GPT)
        lax.fori_loop(0, GPT, group_step, lo0)

    return body


def _cnt_body():
    def body(lidxp, goffs, zeros, agg, lidxv, acc, offv, sem):
        c = lax.axis_index("c")
        s = lax.axis_index("s")
        w = c * 16 + s

        pltpu.sync_copy(goffs, offv)
        pltpu.sync_copy(zeros, acc)
        ones = jnp.full((16,), 1.0, jnp.float32)

        def read_off(i):
            return offv[pl.ds(i, 16)][0]

        def group_step(j, lo):
            g = w * GPT + j
            hi = read_off(g + 1)
            lo_al = (lo // 8) * 8
            nb = (hi - lo_al + CHE - 1) // CHE

            def batch(bi, _):
                abase = lo_al + bi * CHE
                pltpu.sync_copy(lidxp.at[pl.ds(abase, CHE)],
                                lidxv.at[pl.ds(0, CHE)])
                e0 = jnp.maximum(lo - abase, 0)
                e1 = jnp.minimum(hi - abase, CHE)

                def edge(e, _):
                    lid = lidxv[pl.ds(e, 16)][0]
                    plsc.addupdate(acc.at[pl.ds(lid * 16, 16)], ones)
                    return 0

                lax.fori_loop(e0, e1, edge, 0)
                return 0

            lax.fori_loop(0, nb, batch, 0)
            pltpu.sync_copy(acc, agg.at[pl.ds(g * ACCR * 16, ACCR * 16)])
            pltpu.sync_copy(zeros, acc)
            return hi

        lo0 = read_off(w * GPT)
        lax.fori_loop(0, GPT, group_step, lo0)

    return body


@functools.cache
def _agg_fn(din, with_cnt=False):
    scratch = [
        pltpu.VMEM((CHE,), jnp.int32),
        pltpu.VMEM((CHE + 16,), jnp.int32),
        pltpu.VMEM((KP, din), jnp.float32),
        pltpu.VMEM((KP, din), jnp.float32),
        pltpu.VMEM((ACCR * din,), jnp.float32),
    ]
    out_type = jax.ShapeDtypeStruct((NPAD * NREL * din,), jnp.float32)
    if with_cnt:
        scratch.append(pltpu.VMEM((ACCR * 16,), jnp.float32))
        out_type = (out_type,
                    jax.ShapeDtypeStruct((NPAD * NREL * 16,), jnp.float32))
    scratch += [
        pltpu.VMEM((GOFFPAD,), jnp.int32),
        pltpu.SemaphoreType.DMA,
        pltpu.SemaphoreType.DMA,
    ]
    return pl.kernel(
        _agg_body(din, with_cnt),
        out_type=out_type,
        mesh=_sc_mesh(),
        scratch_types=scratch,
        name=f"rgcn_sc_agg_{din}" + ("_cnt" if with_cnt else ""),
    )


@functools.cache
def _cnt_fn():
    scratch = [
        pltpu.VMEM((CHE + 16,), jnp.int32),
        pltpu.VMEM((ACCR * 16,), jnp.float32),
        pltpu.VMEM((GOFFPAD,), jnp.int32),
        pltpu.SemaphoreType.DMA,
    ]
    return pl.kernel(
        _cnt_body(),
        out_type=jax.ShapeDtypeStruct((NPAD * NREL * 16,), jnp.float32),
        mesh=_sc_mesh(),
        scratch_types=scratch,
        name="rgcn_sc_cnt",
    )


@functools.cache
def _w_fn(din, dout):
    cols = din * dout
    bw = 8192

    def body(comp_ref, basis_ref, o_ref):
        o_ref[...] = jnp.dot(comp_ref[...], basis_ref[...],
                             preferred_element_type=jnp.float32)

    return pl.pallas_call(
        body,
        grid=(cols // bw,),
        in_specs=[
            pl.BlockSpec((NREL, NBASES), lambda j: (0, 0)),
            pl.BlockSpec((NBASES, bw), lambda j: (0, j)),
        ],
        out_specs=pl.BlockSpec((NREL, bw), lambda j: (0, j)),
        out_shape=jax.ShapeDtypeStruct((NREL, cols), jnp.float32),
        name=f"rgcn_tc_w_{din}_{dout}",
    )


@functools.cache
def _main_fn(din, dout):
    def body(h_ref, root_ref, w_ref, agg_ref, cnt_ref, bias_ref, o_ref):
        acc = jnp.dot(h_ref[...], root_ref[...],
                      preferred_element_type=jnp.float32)
        parts = []
        for r in range(NREL):
            inv = 1.0 / jnp.maximum(cnt_ref[:, r * 16:r * 16 + 1], 1.0)
            parts.append(agg_ref[:, r * din:(r + 1) * din] * inv)
        sa = jnp.concatenate(parts, axis=1)
        acc = acc + jnp.dot(sa, w_ref[...],
                            preferred_element_type=jnp.float32)
        o_ref[...] = jnp.maximum(acc + bias_ref[...], 0.0)

    return pl.pallas_call(
        body,
        grid=(NBLK,),
        in_specs=[
            pl.BlockSpec((ND, din), lambda i: (i, 0)),
            pl.BlockSpec((din, dout), lambda i: (0, 0)),
            pl.BlockSpec((NREL * din, dout), lambda i: (0, 0)),
            pl.BlockSpec((ND, NREL * din), lambda i: (i, 0)),
            pl.BlockSpec((ND, NREL * 16), lambda i: (i, 0)),
            pl.BlockSpec((1, dout), lambda i: (0, 0)),
        ],
        out_specs=pl.BlockSpec((ND, dout), lambda i: (i, 0)),
        out_shape=jax.ShapeDtypeStruct((NPAD, dout), jnp.float32),
        name=f"rgcn_tc_main_{din}_{dout}",
    )


def kernel(x, edge_index, edge_type,
           basis0, comp0, root0, bias0,
           basis1, comp1, root1, bias1,
           basis2, comp2, root2, bias2):
    src = edge_index[0]
    dst = edge_index[1]
    key = (jnp.left_shift(dst.astype(jnp.uint32), 17)
           | jnp.left_shift(edge_type.astype(jnp.uint32), 14)
           | src.astype(jnp.uint32))
    key = jnp.sort(key)
    src_s = jnp.bitwise_and(key, (1 << 14) - 1).astype(jnp.int32)
    lidx = jnp.bitwise_and(jnp.right_shift(key, 14), 127).astype(jnp.int32)
    pad = EPAD - N_EDGES
    srcp = jnp.concatenate([src_s, jnp.zeros((pad,), jnp.int32)])
    lidxp = jnp.concatenate([lidx, jnp.zeros((pad,), jnp.int32)])
    goffs = jnp.searchsorted(
        key,
        jnp.left_shift(jnp.arange(NGROUPS + 1, dtype=jnp.uint32) * GPN,
                       17)).astype(jnp.int32)
    goffs = jnp.concatenate(
        [goffs, jnp.zeros((GOFFPAD - NGROUPS - 1,), jnp.int32)])

    h = jnp.pad(x, ((0, NPAD - N_NODES), (0, 0)))
    params = [(basis0, comp0, root0, bias0),
              (basis1, comp1, root1, bias1),
              (basis2, comp2, root2, bias2)]
    cnt = None
    for li, (basis, comp, root, bias) in enumerate(params):
        din, dout = basis.shape[1], basis.shape[2]
        if li == 0:
            aggflat, cntflat = _agg_fn(din, True)(
                h, srcp, lidxp, goffs,
                jnp.zeros((ACCR * din,), jnp.float32),
                jnp.zeros((ACCR * 16,), jnp.float32))
            cnt = cntflat.reshape(NPAD, NREL * 16)
        else:
            aggflat = _agg_fn(din)(h, srcp, lidxp, goffs,
                                   jnp.zeros((ACCR * din,), jnp.float32))
        agg = aggflat.reshape(NPAD, NREL * din)
        w2 = _w_fn(din, dout)(comp, basis.reshape(NBASES, din * dout))
        wstk = w2.reshape(NREL * din, dout)
        h = _main_fn(din, dout)(h, root, wstk, agg, cnt,
                                bias.reshape(1, dout))
    return h[:N_NODES]


# speedup vs baseline: 1.0275x; 1.0154x over previous
import functools

import jax
import jax.numpy as jnp
from jax import lax
from jax.experimental import pallas as pl
from jax.experimental.pallas import tpu as pltpu
from jax.experimental.pallas import tpu_sc as plsc

N_NODES = 10000
N_EDGES = 160000
NREL = 8
NBASES = 30

NTILES = 32
GPN = 16
NGROUPS = 640
NPAD = NGROUPS * GPN
GPT = NGROUPS // NTILES
ACCR = GPN * NREL
KP = 48
NBCH = 8
CHE = NBCH * KP
EPAD = 160512
GOFFPAD = 656
ND = 512
NBLK = NPAD // ND


def _sc_mesh():
    return plsc.VectorSubcoreMesh(core_axis_name="c", subcore_axis_name="s")


def _agg_body(din, with_cnt=False):
    def body(*refs):
        if with_cnt:
            (h, srcp, lidxp, goffs, zeros, zeros16, agg, cnt,
             srcv, lidxv, stage0, stage1, acc, accc, offv,
             sem0, sem1) = refs
        else:
            (h, srcp, lidxp, goffs, zeros, agg,
             srcv, lidxv, stage0, stage1, acc, offv, sem0, sem1) = refs
            accc = zeros16 = cnt = None
        ones = jnp.full((16,), 1.0, jnp.float32)
        c = lax.axis_index("c")
        s = lax.axis_index("s")
        w = c * 16 + s

        pltpu.sync_copy(goffs, offv)
        pltpu.sync_copy(zeros, acc)
        if with_cnt:
            pltpu.sync_copy(zeros16, accc)
        stages = (stage0, stage1)
        sems = (sem0, sem1)

        def read_off(i):
            return offv[pl.ds(i, 16)][0]

        def group_step(j, lo):
            g = w * GPT + j
            hi = read_off(g + 1)
            lo_al = (lo // KP) * KP
            nb = (hi - lo_al + KP - 1) // KP
            nc = (nb + NBCH - 1) // NBCH

            def chunk_step(ci, _):
                cbase = lo_al + ci * CHE
                nbc = jnp.minimum(nb - ci * NBCH, NBCH)
                pltpu.sync_copy(srcp.at[pl.ds(cbase, CHE)], srcv)
                pltpu.sync_copy(lidxp.at[pl.ds(cbase, CHE)],
                                lidxv.at[pl.ds(0, CHE)])

                def idxr(b):
                    return srcv.at[pl.ds(b * KP, KP)]

                @pl.when(nbc > 0)
                def _():
                    pltpu.async_copy(h.at[idxr(0)], stages[0], sems[0])

                for b in range(NBCH):
                    if b + 1 < NBCH:
                        @pl.when(b + 1 < nbc)
                        def _(b=b):
                            pltpu.async_copy(h.at[idxr(b + 1)],
                                             stages[(b + 1) % 2],
                                             sems[(b + 1) % 2])

                    @pl.when(b < nbc)
                    def _(b=b):
                        pltpu.make_async_copy(h.at[idxr(b)],
                                              stages[b % 2],
                                              sems[b % 2]).wait()
                        bb = cbase + b * KP
                        e0 = jnp.maximum(lo - bb, 0)
                        e1 = jnp.minimum(hi - bb, KP)
                        stg = stages[b % 2]

                        @plsc.parallel_loop(e0, e1, 1, unroll=2)
                        def edge(e):
                            lid = lidxv[pl.ds(b * KP + e, 16)][0]
                            base = lid * din
                            if with_cnt:
                                plsc.addupdate(
                                    accc.at[pl.ds(lid * 16, 16)], ones)
                            for ch in range(din // 16):
                                v = stg[e, pl.ds(ch * 16, 16)]
                                plsc.addupdate(
                                    acc.at[pl.ds(base + ch * 16, 16)], v)
                return 0

            lax.fori_loop(0, nc, chunk_step, 0)
            pltpu.sync_copy(acc, agg.at[pl.ds(g * ACCR * din, ACCR * din)])
            pltpu.sync_copy(zeros, acc)
            if with_cnt:
                pltpu.sync_copy(accc,
                                cnt.at[pl.ds(g * ACCR * 16, ACCR * 16)])
                pltpu.sync_copy(zeros16, accc)
            return hi

        lo0 = read_off(w * GPT)
        lax.fori_loop(0, GPT, group_step, lo0)

    return body


def _cnt_body():
    def body(lidxp, goffs, zeros, agg, lidxv, acc, offv, sem):
        c = lax.axis_index("c")
        s = lax.axis_index("s")
        w = c * 16 + s

        pltpu.sync_copy(goffs, offv)
        pltpu.sync_copy(zeros, acc)
        ones = jnp.full((16,), 1.0, jnp.float32)

        def read_off(i):
            return offv[pl.ds(i, 16)][0]

        def group_step(j, lo):
            g = w * GPT + j
            hi = read_off(g + 1)
            lo_al = (lo // 8) * 8
            nb = (hi - lo_al + CHE - 1) // CHE

            def batch(bi, _):
                abase = lo_al + bi * CHE
                pltpu.sync_copy(lidxp.at[pl.ds(abase, CHE)],
                                lidxv.at[pl.ds(0, CHE)])
                e0 = jnp.maximum(lo - abase, 0)
                e1 = jnp.minimum(hi - abase, CHE)

                def edge(e, _):
                    lid = lidxv[pl.ds(e, 16)][0]
                    plsc.addupdate(acc.at[pl.ds(lid * 16, 16)], ones)
                    return 0

                lax.fori_loop(e0, e1, edge, 0)
                return 0

            lax.fori_loop(0, nb, batch, 0)
            pltpu.sync_copy(acc, agg.at[pl.ds(g * ACCR * 16, ACCR * 16)])
            pltpu.sync_copy(zeros, acc)
            return hi

        lo0 = read_off(w * GPT)
        lax.fori_loop(0, GPT, group_step, lo0)

    return body


@functools.cache
def _agg_fn(din, with_cnt=False):
    scratch = [
        pltpu.VMEM((CHE,), jnp.int32),
        pltpu.VMEM((CHE + 16,), jnp.int32),
        pltpu.VMEM((KP, din), jnp.float32),
        pltpu.VMEM((KP, din), jnp.float32),
        pltpu.VMEM((ACCR * din,), jnp.float32),
    ]
    out_type = jax.ShapeDtypeStruct((NPAD * NREL * din,), jnp.float32)
    if with_cnt:
        scratch.append(pltpu.VMEM((ACCR * 16,), jnp.float32))
        out_type = (out_type,
                    jax.ShapeDtypeStruct((NPAD * NREL * 16,), jnp.float32))
    scratch += [
        pltpu.VMEM((GOFFPAD,), jnp.int32),
        pltpu.SemaphoreType.DMA,
        pltpu.SemaphoreType.DMA,
    ]
    return pl.kernel(
        _agg_body(din, with_cnt),
        out_type=out_type,
        mesh=_sc_mesh(),
        scratch_types=scratch,
        name=f"rgcn_sc_agg_{din}" + ("_cnt" if with_cnt else ""),
    )


@functools.cache
def _cnt_fn():
    scratch = [
        pltpu.VMEM((CHE + 16,), jnp.int32),
        pltpu.VMEM((ACCR * 16,), jnp.float32),
        pltpu.VMEM((GOFFPAD,), jnp.int32),
        pltpu.SemaphoreType.DMA,
    ]
    return pl.kernel(
        _cnt_body(),
        out_type=jax.ShapeDtypeStruct((NPAD * NREL * 16,), jnp.float32),
        mesh=_sc_mesh(),
        scratch_types=scratch,
        name="rgcn_sc_cnt",
    )


@functools.cache
def _w_fn(din, dout):
    cols = din * dout
    bw = 8192

    def body(comp_ref, basis_ref, o_ref):
        o_ref[...] = jnp.dot(comp_ref[...], basis_ref[...],
                             preferred_element_type=jnp.float32)

    return pl.pallas_call(
        body,
        grid=(cols // bw,),
        in_specs=[
            pl.BlockSpec((NREL, NBASES), lambda j: (0, 0)),
            pl.BlockSpec((NBASES, bw), lambda j: (0, j)),
        ],
        out_specs=pl.BlockSpec((NREL, bw), lambda j: (0, j)),
        out_shape=jax.ShapeDtypeStruct((NREL, cols), jnp.float32),
        name=f"rgcn_tc_w_{din}_{dout}",
    )


@functools.cache
def _main_fn(din, dout):
    def body(h_ref, root_ref, w_ref, agg_ref, cnt_ref, bias_ref, o_ref):
        acc = jnp.dot(h_ref[...], root_ref[...],
                      preferred_element_type=jnp.float32)
        parts = []
        for r in range(NREL):
            inv = 1.0 / jnp.maximum(cnt_ref[:, r * 16:r * 16 + 1], 1.0)
            parts.append(agg_ref[:, r * din:(r + 1) * din] * inv)
        sa = jnp.concatenate(parts, axis=1)
        acc = acc + jnp.dot(sa, w_ref[...],
                            preferred_element_type=jnp.float32)
        o_ref[...] = jnp.maximum(acc + bias_ref[...], 0.0)

    return pl.pallas_call(
        body,
        grid=(NBLK,),
        in_specs=[
            pl.BlockSpec((ND, din), lambda i: (i, 0)),
            pl.BlockSpec((din, dout), lambda i: (0, 0)),
            pl.BlockSpec((NREL * din, dout), lambda i: (0, 0)),
            pl.BlockSpec((ND, NREL * din), lambda i: (i, 0)),
            pl.BlockSpec((ND, NREL * 16), lambda i: (i, 0)),
            pl.BlockSpec((1, dout), lambda i: (0, 0)),
        ],
        out_specs=pl.BlockSpec((ND, dout), lambda i: (i, 0)),
        out_shape=jax.ShapeDtypeStruct((NPAD, dout), jnp.float32),
        name=f"rgcn_tc_main_{din}_{dout}",
    )


def kernel(x, edge_index, edge_type,
           basis0, comp0, root0, bias0,
           basis1, comp1, root1, bias1,
           basis2, comp2, root2, bias2):
    src = edge_index[0]
    dst = edge_index[1]
    key = (jnp.left_shift(dst.astype(jnp.uint32), 17)
           | jnp.left_shift(edge_type.astype(jnp.uint32), 14)
           | src.astype(jnp.uint32))
    key = jnp.sort(key)
    src_s = jnp.bitwise_and(key, (1 << 14) - 1).astype(jnp.int32)
    lidx = jnp.bitwise_and(jnp.right_shift(key, 14), 127).astype(jnp.int32)
    pad = EPAD - N_EDGES
    srcp = jnp.concatenate([src_s, jnp.zeros((pad,), jnp.int32)])
    lidxp = jnp.concatenate([lidx, jnp.zeros((pad,), jnp.int32)])
    goffs = jnp.searchsorted(
        key,
        jnp.left_shift(jnp.arange(NGROUPS + 1, dtype=jnp.uint32) * GPN,
                       17)).astype(jnp.int32)
    goffs = jnp.concatenate(
        [goffs, jnp.zeros((GOFFPAD - NGROUPS - 1,), jnp.int32)])

    h = jnp.pad(x, ((0, NPAD - N_NODES), (0, 0)))
    params = [(basis0, comp0, root0, bias0),
              (basis1, comp1, root1, bias1),
              (basis2, comp2, root2, bias2)]
    cnt = None
    for li, (basis, comp, root, bias) in enumerate(params):
        din, dout = basis.shape[1], basis.shape[2]
        w2 = _w_fn(din, dout)(comp, basis.reshape(NBASES, din * dout))
        wstk = w2.reshape(NREL * din, dout)
        if li == 0:
            aggflat, cntflat = _agg_fn(din, True)(
                h, srcp, lidxp, goffs,
                jnp.zeros((ACCR * din,), jnp.float32),
                jnp.zeros((ACCR * 16,), jnp.float32))
            cnt = cntflat.reshape(NPAD, NREL * 16)
        else:
            aggflat = _agg_fn(din)(h, srcp, lidxp, goffs,
                                   jnp.zeros((ACCR * din,), jnp.float32))
        agg = aggflat.reshape(NPAD, NREL * din)
        h = _main_fn(din, dout)(h, root, wstk, agg, cnt,
                                bias.reshape(1, dout))
    return h[:N_NODES]
